# baseline (speedup 1.0000x reference)
import functools
import math

import jax
import jax.numpy as jnp
from jax.experimental import pallas as pl
from jax.experimental.pallas import tpu as pltpu

_INTERPRET = False

HID = 1024
INTER = 4096
NE = 8
S = 2048
LANES = 128
BLK = 256
NB = S * 2 // BLK + NE
RPAD = NB * BLK


def _gate_body(x_ref, w_ref, b_ref, gw_ref, i1_ref, i2_ref):
    lg = jnp.dot(x_ref[...], w_ref[...], preferred_element_type=jnp.float32)
    lg = lg + b_ref[...]
    lanes = jax.lax.broadcasted_iota(jnp.int32, lg.shape, 1)
    m1 = jnp.max(lg, axis=1, keepdims=True)
    i1 = jnp.min(jnp.where(lg == m1, lanes, LANES), axis=1, keepdims=True)
    lg2 = jnp.where(lanes == i1, -3e38, lg)
    m2 = jnp.max(lg2, axis=1, keepdims=True)
    i2 = jnp.min(jnp.where(lg2 == m2, lanes, LANES), axis=1, keepdims=True)
    e2 = jnp.exp(m2 - m1)
    w1 = 1.0 / (1.0 + e2)
    w2 = 1.0 - w1
    gw = jnp.where(lanes == i1, w1, 0.0) + jnp.where(lanes == i2, w2, 0.0)
    gw_ref[...] = gw
    i1_ref[...] = jnp.broadcast_to(i1, lg.shape)
    i2_ref[...] = jnp.broadcast_to(i2, lg.shape)


def _gate(x2d, gate_W, gate_b):
    gWp = jnp.pad(gate_W, ((0, 0), (0, LANES - NE)))
    gbp = jnp.concatenate(
        [gate_b, jnp.full((LANES - NE,), -1e30, jnp.float32)]).reshape(1, LANES)
    return pl.pallas_call(
        _gate_body,
        out_shape=(
            jax.ShapeDtypeStruct((S, LANES), jnp.float32),
            jax.ShapeDtypeStruct((S, LANES), jnp.int32),
            jax.ShapeDtypeStruct((S, LANES), jnp.int32),
        ),
        interpret=_INTERPRET,
    )(x2d, gWp, gbp)


IC = 4
IQ = INTER // IC


def _gelu(h):
    return 0.5 * h * (1.0 + jax.lax.erf(h * 0.7071067811865476))


def _ffn_body(be_ref, live_ref, xg_ref, w1_ref, b1_ref, w2_ref, b2_ref,
              ws_ref, out_ref, acc_ref):
    ic = pl.program_id(0)
    b = pl.program_id(1)

    @pl.when(live_ref[b] == 1)
    def _():
        rows = pl.ds(b * BLK, BLK)
        xb = xg_ref[rows, :].astype(jnp.float32)
        h = jnp.dot(xb, w1_ref[0], preferred_element_type=jnp.float32)
        h = _gelu(h + b1_ref[0])
        contrib = jnp.dot(h, w2_ref[0], preferred_element_type=jnp.float32)

        @pl.when(ic == 0)
        def _():
            acc_ref[rows, :] = contrib

        @pl.when(ic > 0)
        def _():
            acc_ref[rows, :] = acc_ref[rows, :] + contrib

        @pl.when(ic == IC - 1)
        def _():
            out_ref[...] = (acc_ref[rows, :] + b2_ref[0]) * ws_ref[...]


def _grouped_ffn(xg16, W1, b1, W2, b2, ws, be, live):
    grid_spec = pltpu.PrefetchScalarGridSpec(
        num_scalar_prefetch=2,
        grid=(IC, NB),
        in_specs=[
            pl.BlockSpec((RPAD, HID), lambda ic, b, be, lv: (0, 0)),
            pl.BlockSpec((1, HID, IQ), lambda ic, b, be, lv: (be[b], 0, ic)),
            pl.BlockSpec((1, 1, IQ), lambda ic, b, be, lv: (be[b], 0, ic)),
            pl.BlockSpec((1, IQ, HID), lambda ic, b, be, lv: (be[b], ic, 0)),
            pl.BlockSpec((1, 1, HID), lambda ic, b, be, lv: (be[b], 0, 0)),
            pl.BlockSpec((BLK, 1), lambda ic, b, be, lv: (b, 0)),
        ],
        out_specs=pl.BlockSpec(
            (BLK, HID),
            lambda ic, b, be, lv: (jnp.where(ic == IC - 1, b, 0), 0)),
        scratch_shapes=[pltpu.VMEM((RPAD, HID), jnp.float32)],
    )
    return pl.pallas_call(
        _ffn_body,
        grid_spec=grid_spec,
        out_shape=jax.ShapeDtypeStruct((RPAD, HID), jnp.float32),
        compiler_params=pltpu.CompilerParams(
            dimension_semantics=("arbitrary", "arbitrary"),
            vmem_limit_bytes=63 * 1024 * 1024,
        ),
        interpret=_INTERPRET,
    )(be, live, xg16, W1, b1.reshape(NE, 1, INTER), W2,
      b2.reshape(NE, 1, HID), ws)


def kernel(x, gate_W, gate_b, W1, b1, W2, b2):
    x2d = x[0]
    gwfull, i1b, i2b = _gate(x2d, gate_W, gate_b)
    gw8 = gwfull[:, :NE]
    i1 = i1b[:, 0]
    i2 = i2b[:, 0]

    e_flat = jnp.stack([i1, i2], axis=1).reshape(-1)
    oh = (e_flat[:, None] == jnp.arange(NE)[None, :]).astype(jnp.int32)
    rank_all = jnp.cumsum(oh, axis=0) - oh
    rank = jnp.take_along_axis(rank_all, e_flat[:, None], axis=1)[:, 0]
    counts = jnp.sum(oh, axis=0)
    pc = ((counts + BLK - 1) // BLK) * BLK
    pbase = jnp.concatenate([jnp.zeros((1,), jnp.int32),
                             jnp.cumsum(pc)[:-1].astype(jnp.int32)])
    padded_pos = pbase[e_flat] + rank
    src_token = jnp.arange(2 * S, dtype=jnp.int32) // 2
    gidx = jnp.zeros((RPAD,), jnp.int32).at[padded_pos].set(src_token)
    w_flat = jnp.take_along_axis(gw8, e_flat.reshape(S, 2), axis=1).reshape(-1)
    ws = jnp.zeros((RPAD,), jnp.float32).at[padded_pos].set(w_flat)
    p0 = padded_pos[0::2]
    p1 = padded_pos[1::2]
    nb_e = pc // BLK
    cumnb = jnp.cumsum(nb_e)
    n_live = cumnb[-1]
    live = (jnp.arange(NB) < n_live).astype(jnp.int32)
    e_last = jnp.max(jnp.where(pc > 0, jnp.arange(NE), 0)).astype(jnp.int32)
    be = jnp.where(
        live == 1,
        jnp.searchsorted(cumnb, jnp.arange(NB), side="right"),
        e_last,
    ).astype(jnp.int32)

    xg16 = jnp.take(x2d.astype(jnp.bfloat16), gidx, axis=0)

    yg = _grouped_ffn(xg16, W1, b1, W2, b2, ws.reshape(RPAD, 1), be, live)

    out = jnp.take(yg, p0, axis=0) + jnp.take(yg, p1, axis=0)
    return (out[None], gw8[None])
